# baseline (speedup 1.0000x reference)
import functools

import jax
import jax.numpy as jnp
from jax import lax
from jax.experimental import pallas as pl
from jax.experimental.pallas import tpu as pltpu
from jax.experimental.pallas import tpu_sc as plsc

N = 10000
NPAD = 10008
F = 128
G = 40
NPG = 250
E = 320000
NW = 32
EPR = E // NW
K = 128
NROW = 80
EPW = NROW * K
VCOL = G
PADW = N * F + VCOL
RING = 2


def _tc_body(x_ref, wgcn_ref, wgraph_ref, wfinal_ref, bgcn_ref, bgraph_ref,
             consts_ref, t_ref, v_ref, p_ref):
    inv = 1.0 / consts_ref[0, 0]
    h = lax.dot_general(x_ref[...], wgcn_ref[...], (((1,), (0,)), ((), ())),
                        preferred_element_type=jnp.float32) * inv
    t_ref[pl.ds(0, N), pl.ds(0, G)] = h
    v_row = lax.dot_general(wfinal_ref[...], wgraph_ref[...],
                            (((1,), (0,)), ((), ())),
                            preferred_element_type=jnp.float32)
    v_ref[0:1, pl.ds(0, N)] = v_row
    v_ref[0:1, pl.ds(N, 16)] = jnp.zeros((1, 16), jnp.float32)
    rows = lax.broadcasted_iota(jnp.int32, (N, G), 0)
    cols = lax.broadcasted_iota(jnp.int32, (N, G), 1)
    mask = (rows >= cols * NPG) & (rows < (cols + 1) * NPG)
    hb = h + jnp.reshape(bgcn_ref[...], (1, G))
    sel = jnp.sum(jnp.where(mask, hb, 0.0), axis=1, keepdims=True)
    t12 = lax.dot_general(v_row, sel, (((1,), (0,)), ((), ())),
                          preferred_element_type=jnp.float32)
    wf = wfinal_ref[...]
    const2 = (jnp.sum(wf * jnp.reshape(bgraph_ref[...], (1, G)))
              - consts_ref[0, 1] * jnp.sum(wf))
    p_ref[...] = t12 + const2


_tc_call = pl.pallas_call(
    _tc_body,
    out_shape=(
        jax.ShapeDtypeStruct((NPAD, F), jnp.float32),
        jax.ShapeDtypeStruct((1, N + 16), jnp.float32),
        jax.ShapeDtypeStruct((1, 1), jnp.float32),
    ),
)


def _sc_body(t_hbm, v_hbm, ei_hbm, out_hbm,
             src_v, dst_v, fh_v, fw_v, slab_h, slab_w, acc_v, *sems):
    wid = lax.axis_index("s") * 2 + lax.axis_index("c")
    pltpu.sync_copy(ei_hbm.at[pl.ds(wid * EPR, EPR)],
                    src_v.at[pl.ds(0, EPR)])
    pltpu.sync_copy(ei_hbm.at[pl.ds(E + wid * EPR, EPR)],
                    dst_v.at[pl.ds(0, EPR)])

    def idx_body(i, carry):
        j = lax.shift_right_logical(i, 3)
        o = lax.bitwise_and(i, 7) * 16
        e16 = i * 16 + lax.iota(jnp.int32, 16)
        valid = e16 < EPR
        s16 = src_v[pl.ds(i * 16, 16)]
        d16 = dst_v[pl.ds(i * 16, 16)]
        c16 = lax.shift_right_logical(d16 * 8389, 21)
        fh = jnp.where(valid, s16 * F + c16, 0)
        fw = jnp.where(valid, d16, N)
        fh_v[j, pl.ds(o, 16)] = fh
        fw_v[j, pl.ds(o, 16)] = fw
        return carry

    lax.fori_loop(0, EPW // 16, idx_body, 0)

    def issue_body(j, carry):
        pltpu.async_copy(t_hbm.at[fh_v.at[j]], slab_h.at[j], sems[0])
        pltpu.async_copy(v_hbm.at[fw_v.at[j]], slab_w.at[j], sems[1])
        return carry

    lax.fori_loop(0, NROW, issue_body, 0)

    def drain_body(j, carry):
        pltpu.make_async_copy(t_hbm.at[fh_v.at[j]], slab_h.at[j],
                              sems[0]).wait()
        pltpu.make_async_copy(v_hbm.at[fw_v.at[j]], slab_w.at[j],
                              sems[1]).wait()
        return carry

    lax.fori_loop(0, NROW, drain_body, 0)

    def fma_body(i, a):
        j = lax.shift_right_logical(i, 3)
        o = lax.bitwise_and(i, 7) * 16
        return (a + slab_h[j, pl.ds(o, 16)] * slab_w[j, pl.ds(o, 16)])

    acc_v[...] = lax.fori_loop(0, EPW // 16, fma_body,
                               jnp.zeros((16,), jnp.float32))
    pltpu.sync_copy(acc_v, out_hbm.at[wid])


@functools.cache
def _get_sc_call():
    return functools.partial(
        pl.kernel,
        mesh=plsc.VectorSubcoreMesh(core_axis_name="c", subcore_axis_name="s"),
        out_type=jax.ShapeDtypeStruct((NW, 16), jnp.float32),
        scratch_types=[
            pltpu.VMEM((EPW,), jnp.int32),
            pltpu.VMEM((EPW,), jnp.int32),
            pltpu.VMEM((NROW, K), jnp.int32),
            pltpu.VMEM((NROW, K), jnp.int32),
            pltpu.VMEM((NROW, K), jnp.float32),
            pltpu.VMEM((NROW, K), jnp.float32),
            pltpu.VMEM((16,), jnp.float32),
        ] + [pltpu.SemaphoreType.DMA] * 2,
    )(_sc_body)


def kernel(x, old_edge_index, W_gcn, b_gcn, W_graph, b_graph, W_final,
           b_final, node_scales, graph_scales, graph_means, final_scales,
           final_means):
    consts = jnp.stack([jnp.asarray(node_scales, jnp.float32),
                        jnp.asarray(final_means, jnp.float32)]).reshape(1, 2)
    t, v2, p = _tc_call(x, W_gcn, W_graph, W_final, b_gcn, b_graph, consts)
    t_flat = jnp.reshape(t, (NPAD * F,))
    v_flat = jnp.reshape(v2, (N + 16,))
    partials = _get_sc_call()(t_flat, v_flat,
                              jnp.reshape(old_edge_index, (2 * E,)))
    s = p[0, 0] + jnp.sum(partials)
    y = jax.nn.sigmoid(s / final_scales + b_final[0])
    return jnp.reshape(y, (1,))


# speedup vs baseline: 1.1031x; 1.1031x over previous
import functools

import jax
import jax.numpy as jnp
from jax import lax
from jax.experimental import pallas as pl
from jax.experimental.pallas import tpu as pltpu
from jax.experimental.pallas import tpu_sc as plsc

N = 10000
NPAD = 10008
F = 128
G = 40
NPG = 250
E = 320000
NW = 32
EPR = E // NW
K = 128
NROW = 80
EPW = NROW * K
VCOL = G
PADW = N * F + VCOL
RING = 2


def _tc_body(x_ref, wgcn_ref, wgraph_ref, wfinal_ref, bgcn_ref, bgraph_ref,
             consts_ref, t_ref, p_ref):
    inv = 1.0 / consts_ref[0, 0]
    h = lax.dot_general(x_ref[...], wgcn_ref[...], (((1,), (0,)), ((), ())),
                        preferred_element_type=jnp.float32) * inv
    t_ref[pl.ds(0, N), pl.ds(0, G)] = h
    v_col = lax.dot_general(wgraph_ref[...], wfinal_ref[...],
                            (((0,), (1,)), ((), ())),
                            preferred_element_type=jnp.float32)
    t_ref[pl.ds(0, N), pl.ds(VCOL, 1)] = v_col
    t_ref[pl.ds(N, NPAD - N), :] = jnp.zeros((NPAD - N, F), jnp.float32)
    rows = lax.broadcasted_iota(jnp.int32, (N, G), 0)
    cols = lax.broadcasted_iota(jnp.int32, (N, G), 1)
    mask = (rows >= cols * NPG) & (rows < (cols + 1) * NPG)
    hb = h + jnp.reshape(bgcn_ref[...], (1, G))
    sel = jnp.sum(jnp.where(mask, hb, 0.0), axis=1, keepdims=True)
    t12 = lax.dot_general(v_col, sel, (((0,), (0,)), ((), ())),
                          preferred_element_type=jnp.float32)
    wf = wfinal_ref[...]
    const2 = (jnp.sum(wf * jnp.reshape(bgraph_ref[...], (1, G)))
              - consts_ref[0, 1] * jnp.sum(wf))
    p_ref[...] = t12 + const2


_tc_call = pl.pallas_call(
    _tc_body,
    out_shape=(
        jax.ShapeDtypeStruct((NPAD, F), jnp.float32),
        jax.ShapeDtypeStruct((1, 1), jnp.float32),
    ),
)


def _sc_body(t_hbm, ei_hbm, out_hbm,
             src_v, dst_v, fh_v, fw_v, slab_h, slab_w, acc_v, *sems):
    wid = lax.axis_index("s") * 2 + lax.axis_index("c")
    pltpu.sync_copy(ei_hbm.at[pl.ds(wid * EPR, EPR)],
                    src_v.at[pl.ds(0, EPR)])
    pltpu.sync_copy(ei_hbm.at[pl.ds(E + wid * EPR, EPR)],
                    dst_v.at[pl.ds(0, EPR)])

    def idx_body(i, carry):
        j = lax.shift_right_logical(i, 3)
        o = lax.bitwise_and(i, 7) * 16
        e16 = i * 16 + lax.iota(jnp.int32, 16)
        valid = e16 < EPR
        s16 = src_v[pl.ds(i * 16, 16)]
        d16 = dst_v[pl.ds(i * 16, 16)]
        c16 = lax.shift_right_logical(d16 * 8389, 21)
        fh = jnp.where(valid, s16 * F + c16, 0)
        fw = jnp.where(valid, d16 * F + VCOL, PADW)
        fh_v[j, pl.ds(o, 16)] = fh
        fw_v[j, pl.ds(o, 16)] = fw
        return carry

    lax.fori_loop(0, EPW // 16, idx_body, 0)

    def issue_body(j, carry):
        pltpu.async_copy(t_hbm.at[fh_v.at[j]], slab_h.at[j], sems[0])
        pltpu.async_copy(t_hbm.at[fw_v.at[j]], slab_w.at[j], sems[1])
        return carry

    lax.fori_loop(0, NROW, issue_body, 0)

    def drain_body(j, carry):
        pltpu.make_async_copy(t_hbm.at[fh_v.at[j]], slab_h.at[j],
                              sems[0]).wait()
        pltpu.make_async_copy(t_hbm.at[fw_v.at[j]], slab_w.at[j],
                              sems[1]).wait()
        return carry

    lax.fori_loop(0, NROW, drain_body, 0)

    def fma_body(i, a):
        j = lax.shift_right_logical(i, 3)
        o = lax.bitwise_and(i, 7) * 16
        return (a + slab_h[j, pl.ds(o, 16)] * slab_w[j, pl.ds(o, 16)])

    acc_v[...] = lax.fori_loop(0, EPW // 16, fma_body,
                               jnp.zeros((16,), jnp.float32))
    pltpu.sync_copy(acc_v, out_hbm.at[wid])


@functools.cache
def _get_sc_call():
    return functools.partial(
        pl.kernel,
        mesh=plsc.VectorSubcoreMesh(core_axis_name="c", subcore_axis_name="s"),
        out_type=jax.ShapeDtypeStruct((NW, 16), jnp.float32),
        scratch_types=[
            pltpu.VMEM((EPW,), jnp.int32),
            pltpu.VMEM((EPW,), jnp.int32),
            pltpu.VMEM((NROW, K), jnp.int32),
            pltpu.VMEM((NROW, K), jnp.int32),
            pltpu.VMEM((NROW, K), jnp.float32),
            pltpu.VMEM((NROW, K), jnp.float32),
            pltpu.VMEM((16,), jnp.float32),
        ] + [pltpu.SemaphoreType.DMA] * 2,
    )(_sc_body)


def kernel(x, old_edge_index, W_gcn, b_gcn, W_graph, b_graph, W_final,
           b_final, node_scales, graph_scales, graph_means, final_scales,
           final_means):
    consts = jnp.stack([jnp.asarray(node_scales, jnp.float32),
                        jnp.asarray(final_means, jnp.float32)]).reshape(1, 2)
    t, p = _tc_call(x, W_gcn, W_graph, W_final, b_gcn, b_graph, consts)
    t_flat = jnp.reshape(t, (NPAD * F,))
    partials = _get_sc_call()(t_flat, jnp.reshape(old_edge_index, (2 * E,)))
    s = p[0, 0] + jnp.sum(partials)
    y = jax.nn.sigmoid(s / final_scales + b_final[0])
    return jnp.reshape(y, (1,))
